# baseline (speedup 1.0000x reference)
import jax
import jax.numpy as jnp
from jax.experimental import pallas as pl
from jax.experimental.pallas import tpu as pltpu

B = 64
V = 100000
D = 128
MAX_SEQ = 2048
CTX = 1
MAX_GEN = MAX_SEQ - CTX
PAD = 0
EOS = 2

VC = 2048
NCHUNK = (V + VC - 1) // VC


def _body(s_ref, cur_vec_ref, emb_ref, w_ref, b_ref, gen_ref,
          tok_out, buf_out, step_out, h_ref, bv_ref, bi_ref, sem):
    i = pl.program_id(0)

    @pl.when(i == 0)
    def _init_and_gather():
        bv_ref[:] = jnp.full((B, 1), -jnp.inf, dtype=jnp.float32)
        bi_ref[:] = jnp.zeros((B, 1), dtype=jnp.int32)

        def _start(r, c):
            idx = s_ref[r]
            pltpu.make_async_copy(
                emb_ref.at[pl.ds(idx, 1), :], h_ref.at[pl.ds(r, 1), :], sem
            ).start()
            return c

        jax.lax.fori_loop(0, B, _start, 0)

        def _wait(r, c):
            idx = s_ref[r]
            pltpu.make_async_copy(
                emb_ref.at[pl.ds(idx, 1), :], h_ref.at[pl.ds(r, 1), :], sem
            ).wait()
            return c

        jax.lax.fori_loop(0, B, _wait, 0)

    logits = jnp.dot(h_ref[:], w_ref[:], preferred_element_type=jnp.float32)
    logits = logits + b_ref[0, :][None, :]
    base = i * VC
    col_ids = base + jax.lax.broadcasted_iota(jnp.int32, (1, VC), 1)
    valid = col_ids < V
    logits = jnp.where(valid, logits, -jnp.inf)
    cmax = jnp.max(logits, axis=1, keepdims=True)
    carg = jnp.min(jnp.where(logits == cmax, col_ids, V), axis=1, keepdims=True)
    upd = cmax > bv_ref[:]
    bi_ref[:] = jnp.where(upd, carg.astype(jnp.int32), bi_ref[:])
    bv_ref[:] = jnp.where(upd, cmax, bv_ref[:])

    @pl.when(i == NCHUNK - 1)
    def _finish():
        cur = cur_vec_ref[:]
        tok = jnp.where(cur == EOS, EOS, bi_ref[:])
        tok_out[:] = tok
        col = s_ref[B]
        begin_new = s_ref[B + 1]
        keep = 1.0 - begin_new.astype(jnp.float32)
        base_buf = gen_ref[:] * keep + (1.0 - keep) * jnp.float32(PAD)
        cids = jax.lax.broadcasted_iota(jnp.int32, (B, MAX_GEN), 1)
        add = jnp.where(cids == col, tok.astype(jnp.float32) - jnp.float32(PAD), 0.0)
        buf_out[:] = base_buf + add
        step_out[0] = col.astype(jnp.float32) + 1.0


def kernel(decoder_input_ids, emb, W_out, b_out, generated_tokens, generation_step):
    stepf = generation_step[0]
    stepc = jnp.where(stepf < MAX_GEN, stepf, 0.0)
    begin_new = (stepc == 0.0).astype(jnp.int32)
    col = stepc.astype(jnp.int32)
    prev_col = jnp.maximum(col - 1, 0)
    prev = jax.lax.dynamic_slice(generated_tokens, (0, prev_col), (B, 1))
    cur = jnp.where(begin_new == 1, decoder_input_ids[:, 0],
                    prev[:, 0].astype(jnp.int32))
    scalars = jnp.concatenate([cur, col[None], begin_new[None]])
    cur_vec = cur[:, None]
    b2 = b_out.reshape(1, V)

    grid_spec = pltpu.PrefetchScalarGridSpec(
        num_scalar_prefetch=1,
        grid=(NCHUNK,),
        in_specs=[
            pl.BlockSpec((B, 1), lambda i, s: (0, 0)),
            pl.BlockSpec(memory_space=pltpu.HBM),
            pl.BlockSpec((D, VC), lambda i, s: (0, i)),
            pl.BlockSpec((1, VC), lambda i, s: (0, i)),
            pl.BlockSpec((B, MAX_GEN), lambda i, s: (0, 0)),
        ],
        out_specs=[
            pl.BlockSpec((B, 1), lambda i, s: (0, 0)),
            pl.BlockSpec((B, MAX_GEN), lambda i, s: (0, 0)),
            pl.BlockSpec(memory_space=pltpu.SMEM),
        ],
        scratch_shapes=[
            pltpu.VMEM((B, D), jnp.float32),
            pltpu.VMEM((B, 1), jnp.float32),
            pltpu.VMEM((B, 1), jnp.int32),
            pltpu.SemaphoreType.DMA,
        ],
    )

    tokens, new_buffer, new_step = pl.pallas_call(
        _body,
        grid_spec=grid_spec,
        out_shape=[
            jax.ShapeDtypeStruct((B, 1), jnp.int32),
            jax.ShapeDtypeStruct((B, MAX_GEN), jnp.float32),
            jax.ShapeDtypeStruct((1,), jnp.float32),
        ],
        compiler_params=pltpu.CompilerParams(
            dimension_semantics=("arbitrary",),
        ),
    )(scalars, cur_vec, emb, W_out, b2, generated_tokens)
    return tokens, new_buffer, new_step


# speedup vs baseline: 1.2764x; 1.2764x over previous
import jax
import jax.numpy as jnp
from jax.experimental import pallas as pl
from jax.experimental.pallas import tpu as pltpu

B = 64
V = 100000
D = 128
MAX_SEQ = 2048
CTX = 1
MAX_GEN = MAX_SEQ - CTX
PAD = 0
EOS = 2

VC = 8192
NCHUNK = (V + VC - 1) // VC


def _body(s_ref, cur_vec_ref, emb_ref, w_ref, b_ref, gen_ref,
          tok_out, buf_out, step_out, h_ref, bv_ref, bi_ref, sem):
    i = pl.program_id(0)

    @pl.when(i == 0)
    def _init_and_gather():
        bv_ref[:] = jnp.full((B, 1), -jnp.inf, dtype=jnp.float32)
        bi_ref[:] = jnp.zeros((B, 1), dtype=jnp.int32)

        def _start(r, c):
            idx = s_ref[r]
            pltpu.make_async_copy(
                emb_ref.at[pl.ds(idx, 1), :], h_ref.at[pl.ds(r, 1), :], sem
            ).start()
            return c

        jax.lax.fori_loop(0, B, _start, 0)

        def _wait(r, c):
            idx = s_ref[r]
            pltpu.make_async_copy(
                emb_ref.at[pl.ds(idx, 1), :], h_ref.at[pl.ds(r, 1), :], sem
            ).wait()
            return c

        jax.lax.fori_loop(0, B, _wait, 0)

    logits = jnp.dot(h_ref[:], w_ref[:], preferred_element_type=jnp.float32)
    logits = logits + b_ref[0, :][None, :]
    base = i * VC
    col_ids = base + jax.lax.broadcasted_iota(jnp.int32, (1, VC), 1)
    valid = col_ids < V
    logits = jnp.where(valid, logits, -jnp.inf)
    cmax = jnp.max(logits, axis=1, keepdims=True)
    carg = jnp.min(jnp.where(logits == cmax, col_ids, V), axis=1, keepdims=True)
    upd = cmax > bv_ref[:]
    bi_ref[:] = jnp.where(upd, carg.astype(jnp.int32), bi_ref[:])
    bv_ref[:] = jnp.where(upd, cmax, bv_ref[:])

    @pl.when(i == NCHUNK - 1)
    def _finish():
        cur = cur_vec_ref[:]
        tok = jnp.where(cur == EOS, EOS, bi_ref[:])
        tok_out[:] = tok
        col = s_ref[B]
        begin_new = s_ref[B + 1]
        keep = 1.0 - begin_new.astype(jnp.float32)
        base_buf = gen_ref[:] * keep + (1.0 - keep) * jnp.float32(PAD)
        cids = jax.lax.broadcasted_iota(jnp.int32, (B, MAX_GEN), 1)
        add = jnp.where(cids == col, tok.astype(jnp.float32) - jnp.float32(PAD), 0.0)
        buf_out[:] = base_buf + add
        step_out[0] = col.astype(jnp.float32) + 1.0


def kernel(decoder_input_ids, emb, W_out, b_out, generated_tokens, generation_step):
    stepf = generation_step[0]
    stepc = jnp.where(stepf < MAX_GEN, stepf, 0.0)
    begin_new = (stepc == 0.0).astype(jnp.int32)
    col = stepc.astype(jnp.int32)
    prev_col = jnp.maximum(col - 1, 0)
    prev = jax.lax.dynamic_slice(generated_tokens, (0, prev_col), (B, 1))
    cur = jnp.where(begin_new == 1, decoder_input_ids[:, 0],
                    prev[:, 0].astype(jnp.int32))
    scalars = jnp.concatenate([cur, col[None], begin_new[None]])
    cur_vec = cur[:, None]
    b2 = b_out.reshape(1, V)

    grid_spec = pltpu.PrefetchScalarGridSpec(
        num_scalar_prefetch=1,
        grid=(NCHUNK,),
        in_specs=[
            pl.BlockSpec((B, 1), lambda i, s: (0, 0)),
            pl.BlockSpec(memory_space=pltpu.HBM),
            pl.BlockSpec((D, VC), lambda i, s: (0, i)),
            pl.BlockSpec((1, VC), lambda i, s: (0, i)),
            pl.BlockSpec((B, MAX_GEN), lambda i, s: (0, 0)),
        ],
        out_specs=[
            pl.BlockSpec((B, 1), lambda i, s: (0, 0)),
            pl.BlockSpec((B, MAX_GEN), lambda i, s: (0, 0)),
            pl.BlockSpec(memory_space=pltpu.SMEM),
        ],
        scratch_shapes=[
            pltpu.VMEM((B, D), jnp.float32),
            pltpu.VMEM((B, 1), jnp.float32),
            pltpu.VMEM((B, 1), jnp.int32),
            pltpu.SemaphoreType.DMA,
        ],
    )

    tokens, new_buffer, new_step = pl.pallas_call(
        _body,
        grid_spec=grid_spec,
        out_shape=[
            jax.ShapeDtypeStruct((B, 1), jnp.int32),
            jax.ShapeDtypeStruct((B, MAX_GEN), jnp.float32),
            jax.ShapeDtypeStruct((1,), jnp.float32),
        ],
        compiler_params=pltpu.CompilerParams(
            dimension_semantics=("arbitrary",),
        ),
    )(scalars, cur_vec, emb, W_out, b2, generated_tokens)
    return tokens, new_buffer, new_step
